# Initial kernel scaffold; baseline (speedup 1.0000x reference)
import jax
import jax.numpy as jnp
from jax.experimental import pallas as pl


def kernel(a_t, b_t, h_t, next_carry_w, digit_w):
    raise NotImplementedError("write your pallas kernel here")


# speedup vs baseline: 2.0552x; 2.0552x over previous
import jax
import jax.numpy as jnp
from jax import lax
from jax.experimental import pallas as pl
from jax.experimental.pallas import tpu as pltpu, tpu_sc as plsc
import functools

_B = 16384
_NC, _NS, _L = 2, 16, 16
_NW = _NC * _NS
_BPW = _B // _NW
_CHUNKS = _BPW // _L


def _body(a_hbm, b_hbm, h_hbm, dw_hbm, cw_hbm, outd_hbm, outc_hbm,
          a_v, b_v, h_v, dw_v, cw_v, outd_v, outc_v):
    wid = lax.axis_index("s") * _NC + lax.axis_index("c")
    base = wid * _BPW

    pltpu.sync_copy(a_hbm.at[pl.ds(base, _BPW)], a_v)
    pltpu.sync_copy(b_hbm.at[pl.ds(base, _BPW)], b_v)
    pltpu.sync_copy(h_hbm.at[pl.ds(2 * base, 2 * _BPW)], h_v)
    pltpu.sync_copy(dw_hbm, dw_v)
    pltpu.sync_copy(cw_hbm, cw_v)

    lane = lax.iota(jnp.int32, _L)
    for c in range(_CHUNKS):
        off = c * _L
        a = a_v[pl.ds(off, _L)]
        b = b_v[pl.ds(off, _L)]
        hpos = lane * 2 + (2 * off)
        h0 = plsc.load_gather(h_v, [hpos])
        h1 = plsc.load_gather(h_v, [hpos + 1])
        carry100 = jnp.where(h1 > h0, 100, 0)
        idx = carry100 + a * 10 + b
        row10 = idx * 10
        out10 = (lane + off) * 10
        for d in range(10):
            col = plsc.load_gather(dw_v, [row10 + d])
            plsc.store_scatter(outd_v, [out10 + d], col)
        row2 = idx * 2
        out2 = (lane + off) * 2
        for d in range(2):
            col = plsc.load_gather(cw_v, [row2 + d])
            plsc.store_scatter(outc_v, [out2 + d], col)

    pltpu.sync_copy(outd_v, outd_hbm.at[pl.ds(base * 10, _BPW * 10)])
    pltpu.sync_copy(outc_v, outc_hbm.at[pl.ds(base * 2, _BPW * 2)])


@jax.jit
def kernel(a_t, b_t, h_t, next_carry_w, digit_w):
    mesh = plsc.VectorSubcoreMesh(
        core_axis_name="c", subcore_axis_name="s",
        num_cores=_NC, num_subcores=_NS)
    run = pl.kernel(
        _body,
        out_type=(
            jax.ShapeDtypeStruct((_B * 10,), jnp.float32),
            jax.ShapeDtypeStruct((_B * 2,), jnp.float32),
        ),
        mesh=mesh,
        compiler_params=pltpu.CompilerParams(needs_layout_passes=False),
        scratch_types=[
            pltpu.VMEM((_BPW,), jnp.int32),
            pltpu.VMEM((_BPW,), jnp.int32),
            pltpu.VMEM((2 * _BPW,), jnp.float32),
            pltpu.VMEM((2000,), jnp.float32),
            pltpu.VMEM((400,), jnp.float32),
            pltpu.VMEM((10 * _BPW,), jnp.float32),
            pltpu.VMEM((2 * _BPW,), jnp.float32),
        ],
    )
    outd, outc = run(
        a_t.astype(jnp.int32),
        b_t.astype(jnp.int32),
        h_t.reshape(-1),
        digit_w.reshape(-1),
        next_carry_w.reshape(-1),
    )
    return outd.reshape(_B, 10), outc.reshape(_B, 2)


# speedup vs baseline: 2.5016x; 1.2172x over previous
import jax
import jax.numpy as jnp
from jax import lax
from jax.experimental import pallas as pl
from jax.experimental.pallas import tpu as pltpu, tpu_sc as plsc

_B = 16384
_NC, _NS, _L = 2, 16, 16
_NW = _NC * _NS
_BPW = _B // _NW
_P = 128
_NPASS = _BPW // _P
_PCHUNKS = _P // _L


def _body(a_hbm, b_hbm, h_hbm, dw_hbm, cw_hbm, outd_hbm, outc_hbm,
          a_v, b_v, h_v, dw_v, cw_v, outd_v, outc_v,
          sem_in, sem_h, sem_out):
    wid = lax.axis_index("s") * _NC + lax.axis_index("c")
    base = wid * _BPW

    cp_a = pltpu.async_copy(a_hbm.at[pl.ds(base, _BPW)], a_v, sem_in)
    cp_b = pltpu.async_copy(b_hbm.at[pl.ds(base, _BPW)], b_v, sem_in)
    cp_dw = pltpu.async_copy(dw_hbm, dw_v, sem_in)
    cp_cw = pltpu.async_copy(cw_hbm, cw_v, sem_in)
    cp_h = pltpu.async_copy(h_hbm.at[pl.ds(base, _P)], h_v, sem_h)
    cp_a.wait()
    cp_b.wait()
    cp_dw.wait()
    cp_cw.wait()

    lane = lax.iota(jnp.int32, _L)
    cols = [jnp.zeros((_L,), jnp.int32) + d for d in range(10)]
    cp_od = cp_oc = None
    for p in range(_NPASS):
        cp_h.wait()
        if cp_od is not None:
            cp_od.wait()
            cp_oc.wait()
        for c in range(_PCHUNKS):
            off = c * _L
            a = a_v[pl.ds(p * _P + off, _L)]
            b = b_v[pl.ds(p * _P + off, _L)]
            row = lane + off
            h0 = plsc.load_gather(h_v, [row, cols[0]])
            h1 = plsc.load_gather(h_v, [row, cols[1]])
            carry100 = jnp.where(h1 > h0, 100, 0)
            idx = carry100 + a * 10 + b
            for d in range(10):
                val = plsc.load_gather(dw_v, [idx, cols[d]])
                plsc.store_scatter(outd_v, [row, cols[d]], val)
            for d in range(2):
                val = plsc.load_gather(cw_v, [idx, cols[d]])
                plsc.store_scatter(outc_v, [row, cols[d]], val)
        if p + 1 < _NPASS:
            cp_h = pltpu.async_copy(
                h_hbm.at[pl.ds(base + (p + 1) * _P, _P)], h_v, sem_h)
        cp_od = pltpu.async_copy(
            outd_v, outd_hbm.at[pl.ds(base + p * _P, _P)], sem_out)
        cp_oc = pltpu.async_copy(
            outc_v, outc_hbm.at[pl.ds(base + p * _P, _P)], sem_out)
    cp_od.wait()
    cp_oc.wait()


@jax.jit
def kernel(a_t, b_t, h_t, next_carry_w, digit_w):
    mesh = plsc.VectorSubcoreMesh(
        core_axis_name="c", subcore_axis_name="s",
        num_cores=_NC, num_subcores=_NS)
    run = pl.kernel(
        _body,
        out_type=(
            jax.ShapeDtypeStruct((_B, 10), jnp.float32),
            jax.ShapeDtypeStruct((_B, 2), jnp.float32),
        ),
        mesh=mesh,
        compiler_params=pltpu.CompilerParams(needs_layout_passes=False),
        scratch_types=[
            pltpu.VMEM((_BPW,), jnp.int32),
            pltpu.VMEM((_BPW,), jnp.int32),
            pltpu.VMEM((_P, 2), jnp.float32),
            pltpu.VMEM((200, 10), jnp.float32),
            pltpu.VMEM((200, 2), jnp.float32),
            pltpu.VMEM((_P, 10), jnp.float32),
            pltpu.VMEM((_P, 2), jnp.float32),
            pltpu.SemaphoreType.DMA,
            pltpu.SemaphoreType.DMA,
            pltpu.SemaphoreType.DMA,
        ],
    )
    return run(a_t.astype(jnp.int32), b_t.astype(jnp.int32),
               h_t, digit_w, next_carry_w)
